# baseline (speedup 1.0000x reference)
import jax
import jax.numpy as jnp
from jax import lax
from jax.experimental import pallas as pl
from jax.experimental.pallas import tpu as pltpu
from jax.experimental.pallas import tpu_sc as plsc

_N = 10000
_D = 128
_E = 320000
_OUT = 64
_NC = 2
_NS = 16
_NW = _NC * _NS
_C = 80
_EPT = _E // _NW
_NCH = _EPT // _C
_G = _C // 16
_DX = 144
_RPT = _N // _NS
_TPAD = 1024
_TPT = _TPAD // _NS


def _sc_body(pki_h, feat_h, tid_h, twg_h, zf_h, zc_h,
             feat_o, cnt_o,
             pki_v, sidxr, didxr, rows_a, rows_b, tid_v, twg_v,
             accf, accc, sem_a, sem_b, sem_ia, sem_ib):
    cid = lax.axis_index("c")
    sid = lax.axis_index("s")
    wid = cid * _NS + sid
    r0 = sid * _RPT
    base = wid * _NCH
    pltpu.sync_copy(zf_h.at[pl.ds(r0, _RPT)], accf.at[pl.ds(r0, _RPT)])
    pltpu.sync_copy(zc_h.at[pl.ds(r0, _RPT)], accc.at[pl.ds(r0, _RPT)])
    pltpu.sync_copy(tid_h.at[pl.ds(sid * _TPT, _TPT)], tid_v.at[0])
    pltpu.sync_copy(twg_h.at[pl.ds(sid * _TPT, _TPT)], twg_v)

    def fetch_idx(j, slot, sem):
        return pltpu.async_copy(pki_h.at[pl.ds(base + j, 1)], pki_v.at[slot], sem)

    def wait_idx(j, slot, sem):
        pltpu.make_async_copy(pki_h.at[pl.ds(base + j, 1)], pki_v.at[slot], sem).wait()

    def unpack(slot):
        for g in range(_G):
            pk = pki_v[slot, 0, pl.ds(16 * g, 16)]
            sidxr[slot, pl.ds(16 * g, 16)] = pk & 0xFFFF
            didxr[slot, pl.ds(16 * g, 16)] = lax.shift_right_logical(pk, 16)

    fetch_idx(0, 0, sem_ia).wait()
    fetch_idx(1, 1, sem_ib).wait()
    unpack(0)
    unpack(1)
    plsc.subcore_barrier()
    pltpu.async_copy(feat_h.at[sidxr.at[0]], rows_a, sem_a)
    pltpu.async_copy(feat_h.at[sidxr.at[1]], rows_b, sem_b)
    fetch_idx(2, 0, sem_ia)
    fetch_idx(3, 1, sem_ib)

    def half_step(j, slot, rows, gsem, isem):
        pltpu.make_async_copy(feat_h.at[sidxr.at[slot]], rows, gsem).wait()
        pltpu.sync_copy(rows, accf.at[didxr.at[slot]], add=True)

        @pl.when(j + 2 < _NCH)
        def _():
            wait_idx(j + 2, slot, isem)
            unpack(slot)

            @pl.when(j + 4 < _NCH)
            def _():
                fetch_idx(j + 4, slot, isem)

            pltpu.async_copy(feat_h.at[sidxr.at[slot]], rows, gsem)

    def step(jj, carry):
        half_step(2 * jj, 0, rows_a, sem_a, sem_ia)
        half_step(2 * jj + 1, 1, rows_b, sem_b, sem_ib)
        return carry

    lax.fori_loop(0, _NCH // 2, step, 0)
    pltpu.make_async_copy(feat_h.at[sidxr.at[0]], rows_a, sem_a).wait()
    pltpu.sync_copy(rows_a, accf.at[didxr.at[0]], add=True)
    pltpu.sync_copy(twg_v, accc.at[tid_v.at[0]], add=True)
    plsc.subcore_barrier()
    o0 = cid * _N + r0
    pltpu.sync_copy(accf.at[pl.ds(r0, _RPT)], feat_o.at[pl.ds(o0, _RPT)])
    pltpu.sync_copy(accc.at[pl.ds(r0, _RPT)], cnt_o.at[pl.ds(o0, _RPT)])


def _tc_body(f0, f1, c0, c1, wp, pr, we, wr, out):
    aug = f0[...] + f1[...]
    agg = aug[:, :_D]
    deg = jnp.maximum(aug[:, _D:_D + 1], 1.0)
    nrm = agg / deg
    h = jnp.maximum(jnp.dot(nrm, wp[...], preferred_element_type=jnp.float32), 0.0)
    wv = c0[:, 1:2] + c1[:, 1:2]
    tpv = jnp.sum(h * wv, axis=0, keepdims=True) / jnp.sum(wv)
    diff = pr[...] - tpv
    d2 = jnp.sum(diff * diff, axis=1, keepdims=True)
    oh = (d2 == jnp.min(d2)).astype(jnp.float32)
    wsel = jnp.sum(we[...] * oh[:, :, None], axis=0)
    x = jnp.maximum(jnp.dot(nrm, wsel, preferred_element_type=jnp.float32), 0.0)
    out[...] = jnp.dot(x, wr[...], preferred_element_type=jnp.float32)


def kernel(features, edge_index, test_ids, W_proj, expert_protos, W_expert, W_reg):
    pki = (edge_index[0] | (edge_index[1] << 16)).reshape(_NW * _NCH, _C)
    featx = jnp.concatenate(
        [features, jnp.ones((_N, 1), jnp.float32),
         jnp.zeros((_N, _DX - _D - 1), jnp.float32)], axis=1)
    ntest = test_ids.shape[0]
    tid_p = jnp.concatenate(
        [test_ids.astype(jnp.int32), jnp.zeros((_TPAD - ntest,), jnp.int32)])
    twg = jnp.zeros((_TPAD, 16), jnp.float32).at[:ntest, 1].set(1.0)
    zf = jnp.zeros((_N, _DX), jnp.float32)
    zc = jnp.zeros((_N, 16), jnp.float32)

    sc_call = pl.kernel(
        _sc_body,
        out_type=[
            jax.ShapeDtypeStruct((_NC * _N, _DX), jnp.float32),
            jax.ShapeDtypeStruct((_NC * _N, 16), jnp.float32),
        ],
        mesh=plsc.VectorSubcoreMesh(core_axis_name="c", subcore_axis_name="s"),
        scratch_types=[
            pltpu.VMEM((2, 1, _C), jnp.int32),
            pltpu.VMEM((2, _C), jnp.int32),
            pltpu.VMEM((2, _C), jnp.int32),
            pltpu.VMEM((_C, _DX), jnp.float32),
            pltpu.VMEM((_C, _DX), jnp.float32),
            pltpu.VMEM((1, _TPT), jnp.int32),
            pltpu.VMEM((_TPT, 16), jnp.float32),
            pltpu.VMEM_SHARED((_N, _DX), jnp.float32),
            pltpu.VMEM_SHARED((_N, 16), jnp.float32),
            pltpu.SemaphoreType.DMA,
            pltpu.SemaphoreType.DMA,
            pltpu.SemaphoreType.DMA,
            pltpu.SemaphoreType.DMA,
        ],
        compiler_params=pltpu.CompilerParams(use_tc_tiling_on_sc=False),
    )
    feat_o, cnt_o = sc_call(pki, featx, tid_p, twg, zf, zc)

    out = pl.pallas_call(
        _tc_body,
        out_shape=jax.ShapeDtypeStruct((_N, _OUT), jnp.float32),
    )(feat_o[:_N], feat_o[_N:], cnt_o[:_N], cnt_o[_N:],
      W_proj, expert_protos, W_expert, W_reg)
    return out


# speedup vs baseline: 1.3222x; 1.3222x over previous
import jax
import jax.numpy as jnp
from jax import lax
from jax.experimental import pallas as pl
from jax.experimental.pallas import tpu as pltpu
from jax.experimental.pallas import tpu_sc as plsc

_N = 10000
_D = 128
_E = 320000
_OUT = 64
_NC = 2
_NS = 16
_NW = _NC * _NS
_C = 80
_EPT = _E // _NW
_NCH = _EPT // _C
_G = _C // 16
_RPT = _N // _NS
_TPAD = 1024
_TPT = _TPAD // _NS


def _sc_body(pki_h, feat_h, tid_h, twg_h, ones_h, zf_h, zc_h,
             feat_o, cnt_o,
             pki_v, sidxr, didxr, rows_a, rows_b, ones_v, tid_v, twg_v,
             accf, accc, sem_a, sem_b):
    cid = lax.axis_index("c")
    sid = lax.axis_index("s")
    wid = cid * _NS + sid
    r0 = sid * _RPT
    pltpu.sync_copy(zf_h.at[pl.ds(r0, _RPT)], accf.at[pl.ds(r0, _RPT)])
    pltpu.sync_copy(zc_h.at[pl.ds(r0, _RPT)], accc.at[pl.ds(r0, _RPT)])
    base = wid * _NCH
    pltpu.sync_copy(pki_h.at[pl.ds(base, _NCH)], pki_v)
    pltpu.sync_copy(ones_h, ones_v)
    pltpu.sync_copy(tid_h.at[pl.ds(sid * _TPT, _TPT)], tid_v.at[0])
    pltpu.sync_copy(twg_h.at[pl.ds(sid * _TPT, _TPT)], twg_v)
    plsc.subcore_barrier()

    def unpack(j, row):
        for g in range(_G):
            pk = pki_v[j, pl.ds(16 * g, 16)]
            sidxr[row, pl.ds(16 * g, 16)] = pk & 0xFFFF
            didxr[row, pl.ds(16 * g, 16)] = lax.shift_right_logical(pk, 16)

    unpack(0, 0)
    unpack(1, 1)
    pltpu.async_copy(feat_h.at[sidxr.at[0]], rows_a, sem_a)
    pltpu.async_copy(feat_h.at[sidxr.at[1]], rows_b, sem_b)

    def step(jj, carry):
        j2 = 2 * jj + 2
        j3 = 2 * jj + 3
        pltpu.make_async_copy(feat_h.at[sidxr.at[0]], rows_a, sem_a).wait()
        pltpu.sync_copy(rows_a, accf.at[didxr.at[0]], add=True)
        pltpu.sync_copy(ones_v, accc.at[didxr.at[0]], add=True)
        unpack(j2, 0)
        pltpu.async_copy(feat_h.at[sidxr.at[0]], rows_a, sem_a)
        pltpu.make_async_copy(feat_h.at[sidxr.at[1]], rows_b, sem_b).wait()
        pltpu.sync_copy(rows_b, accf.at[didxr.at[1]], add=True)
        pltpu.sync_copy(ones_v, accc.at[didxr.at[1]], add=True)

        @pl.when(j3 < _NCH)
        def _():
            unpack(j3, 1)
            pltpu.async_copy(feat_h.at[sidxr.at[1]], rows_b, sem_b)

        return carry

    lax.fori_loop(0, _NCH // 2, step, 0)
    pltpu.make_async_copy(feat_h.at[sidxr.at[0]], rows_a, sem_a).wait()
    pltpu.sync_copy(rows_a, accf.at[didxr.at[0]], add=True)
    pltpu.sync_copy(ones_v, accc.at[didxr.at[0]], add=True)
    pltpu.sync_copy(twg_v, accc.at[tid_v.at[0]], add=True)
    plsc.subcore_barrier()
    o0 = cid * _N + r0
    pltpu.sync_copy(accf.at[pl.ds(r0, _RPT)], feat_o.at[pl.ds(o0, _RPT)])
    pltpu.sync_copy(accc.at[pl.ds(r0, _RPT)], cnt_o.at[pl.ds(o0, _RPT)])


def _tc_body(fp, cp, wp, pr, we, wr, out):
    f = fp[...]
    c = cp[...]
    agg = f[:_N] + f[_N:]
    deg = jnp.maximum(c[:_N, 0:1] + c[_N:, 0:1], 1.0)
    nrm = agg / deg
    h = jnp.maximum(jnp.dot(nrm, wp[...], preferred_element_type=jnp.float32), 0.0)
    wv = c[:_N, 1:2] + c[_N:, 1:2]
    tpv = jnp.sum(h * wv, axis=0, keepdims=True) / jnp.sum(wv)
    diff = pr[...] - tpv
    d2 = jnp.sum(diff * diff, axis=1, keepdims=True)
    oh = (d2 == jnp.min(d2)).astype(jnp.float32)
    wsel = jnp.sum(we[...] * oh[:, :, None], axis=0)
    x = jnp.maximum(jnp.dot(nrm, wsel, preferred_element_type=jnp.float32), 0.0)
    out[...] = jnp.dot(x, wr[...], preferred_element_type=jnp.float32)


def kernel(features, edge_index, test_ids, W_proj, expert_protos, W_expert, W_reg):
    pki = (edge_index[0] | (edge_index[1] << 16)).reshape(_NW * _NCH, _C)
    ntest = test_ids.shape[0]
    tid_p = jnp.concatenate(
        [test_ids.astype(jnp.int32), jnp.zeros((_TPAD - ntest,), jnp.int32)])
    twg = jnp.zeros((_TPAD, 16), jnp.float32).at[:ntest, 1].set(1.0)
    ones_c = jnp.zeros((_C, 16), jnp.float32).at[:, 0].set(1.0)
    zf = jnp.zeros((_N, _D), jnp.float32)
    zc = jnp.zeros((_N, 16), jnp.float32)

    sc_call = pl.kernel(
        _sc_body,
        out_type=[
            jax.ShapeDtypeStruct((_NC * _N, _D), jnp.float32),
            jax.ShapeDtypeStruct((_NC * _N, 16), jnp.float32),
        ],
        mesh=plsc.VectorSubcoreMesh(core_axis_name="c", subcore_axis_name="s"),
        scratch_types=[
            pltpu.VMEM((_NCH, _C), jnp.int32),
            pltpu.VMEM((2, _C), jnp.int32),
            pltpu.VMEM((2, _C), jnp.int32),
            pltpu.VMEM((_C, _D), jnp.float32),
            pltpu.VMEM((_C, _D), jnp.float32),
            pltpu.VMEM((_C, 16), jnp.float32),
            pltpu.VMEM((1, _TPT), jnp.int32),
            pltpu.VMEM((_TPT, 16), jnp.float32),
            pltpu.VMEM_SHARED((_N, _D), jnp.float32),
            pltpu.VMEM_SHARED((_N, 16), jnp.float32),
            pltpu.SemaphoreType.DMA,
            pltpu.SemaphoreType.DMA,
        ],
        compiler_params=pltpu.CompilerParams(use_tc_tiling_on_sc=False),
    )
    feat_o, cnt_o = sc_call(pki, features, tid_p, twg, ones_c, zf, zc)

    out = pl.pallas_call(
        _tc_body,
        out_shape=jax.ShapeDtypeStruct((_N, _OUT), jnp.float32),
    )(feat_o, cnt_o, W_proj, expert_protos, W_expert, W_reg)
    return out


# speedup vs baseline: 1.3224x; 1.0001x over previous
import jax
import jax.numpy as jnp
from jax import lax
from jax.experimental import pallas as pl
from jax.experimental.pallas import tpu as pltpu
from jax.experimental.pallas import tpu_sc as plsc

_N = 10000
_D = 128
_E = 320000
_OUT = 64
_NC = 2
_NS = 16
_NW = _NC * _NS
_C = 80
_EPT = _E // _NW
_NCH = _EPT // _C
_G = _C // 16
_RPT = _N // _NS
_TPAD = 1024
_TPT = _TPAD // _NS


def _sc_body(pki_h, feat_h, tid_h, twg_h, ones_h, zf_h, zc_h,
             feat_o, cnt_o,
             pki_v, sidxr, didxr, rows_a, rows_b, ones_v, tid_v, twg_v,
             accf, accc, sem_a, sem_b):
    cid = lax.axis_index("c")
    sid = lax.axis_index("s")
    wid = cid * _NS + sid
    r0 = sid * _RPT
    pltpu.sync_copy(zf_h.at[pl.ds(r0, _RPT)], accf.at[pl.ds(r0, _RPT)])
    pltpu.sync_copy(zc_h.at[pl.ds(r0, _RPT)], accc.at[pl.ds(r0, _RPT)])
    base = wid * _NCH
    pltpu.sync_copy(pki_h.at[pl.ds(base, _NCH)], pki_v)
    pltpu.sync_copy(ones_h, ones_v)
    pltpu.sync_copy(tid_h.at[pl.ds(sid * _TPT, _TPT)], tid_v.at[0])
    pltpu.sync_copy(twg_h.at[pl.ds(sid * _TPT, _TPT)], twg_v)
    plsc.subcore_barrier()

    def unpack(j, row):
        for g in range(_G):
            pk = pki_v[j, pl.ds(16 * g, 16)]
            sidxr[row, pl.ds(16 * g, 16)] = pk & 0xFFFF
            didxr[row, pl.ds(16 * g, 16)] = lax.shift_right_logical(pk, 16)

    unpack(0, 0)
    unpack(1, 1)
    pltpu.async_copy(feat_h.at[sidxr.at[0]], rows_a, sem_a)
    pltpu.async_copy(feat_h.at[sidxr.at[1]], rows_b, sem_b)

    def step(jj, carry):
        j2 = 2 * jj + 2
        j3 = 2 * jj + 3
        pltpu.make_async_copy(feat_h.at[sidxr.at[0]], rows_a, sem_a).wait()
        pltpu.sync_copy(rows_a, accf.at[didxr.at[0]], add=True)
        pltpu.sync_copy(ones_v, accc.at[didxr.at[0]], add=True)
        unpack(j2, 0)
        pltpu.async_copy(feat_h.at[sidxr.at[0]], rows_a, sem_a)
        pltpu.make_async_copy(feat_h.at[sidxr.at[1]], rows_b, sem_b).wait()
        pltpu.sync_copy(rows_b, accf.at[didxr.at[1]], add=True)
        pltpu.sync_copy(ones_v, accc.at[didxr.at[1]], add=True)

        @pl.when(j3 < _NCH)
        def _():
            unpack(j3, 1)
            pltpu.async_copy(feat_h.at[sidxr.at[1]], rows_b, sem_b)

        return carry

    lax.fori_loop(0, _NCH // 2, step, 0)
    pltpu.make_async_copy(feat_h.at[sidxr.at[0]], rows_a, sem_a).wait()
    pltpu.sync_copy(rows_a, accf.at[didxr.at[0]], add=True)
    pltpu.sync_copy(ones_v, accc.at[didxr.at[0]], add=True)
    pltpu.sync_copy(twg_v, accc.at[tid_v.at[0]], add=True)
    plsc.subcore_barrier()
    o0 = cid * _N + r0
    pltpu.sync_copy(accf.at[pl.ds(r0, _RPT)], feat_o.at[pl.ds(o0, _RPT)])
    pltpu.sync_copy(accc.at[pl.ds(r0, _RPT)], cnt_o.at[pl.ds(o0, _RPT)])


def _tc_body(fp, cp, wp, pr, we, wr, out):
    f = fp[...]
    c = cp[...]
    agg = f[:_N] + f[_N:]
    deg = jnp.maximum(c[:_N, 0:1] + c[_N:, 0:1], 1.0)
    nrm = agg / deg
    h = jnp.maximum(jnp.dot(nrm, wp[...], preferred_element_type=jnp.float32), 0.0)
    wv = c[:_N, 1:2] + c[_N:, 1:2]
    tpv = jnp.sum(h * wv, axis=0, keepdims=True) / jnp.sum(wv)
    diff = pr[...] - tpv
    d2 = jnp.sum(diff * diff, axis=1, keepdims=True)
    oh = (d2 == jnp.min(d2)).astype(jnp.float32)
    wsel = jnp.sum(we[...] * oh[:, :, None], axis=0)
    x = jnp.maximum(jnp.dot(nrm, wsel, preferred_element_type=jnp.float32), 0.0)
    out[...] = jnp.dot(x, wr[...], preferred_element_type=jnp.float32)


def kernel(features, edge_index, test_ids, W_proj, expert_protos, W_expert, W_reg):
    pki = (edge_index[0] | (edge_index[1] << 16)).reshape(_NW * _NCH, _C)
    ntest = test_ids.shape[0]
    tid_p = jnp.concatenate(
        [test_ids.astype(jnp.int32), jnp.zeros((_TPAD - ntest,), jnp.int32)])
    twg = jnp.zeros((_TPAD, 16), jnp.float32).at[:ntest, 1].set(1.0)
    ones_c = jnp.zeros((_C, 16), jnp.float32).at[:, 0].set(1.0)
    zf = jnp.zeros((_N, _D), jnp.float32)
    zc = jnp.zeros((_N, 16), jnp.float32)

    sc_call = pl.kernel(
        _sc_body,
        out_type=[
            jax.ShapeDtypeStruct((_NC * _N, _D), jnp.float32),
            jax.ShapeDtypeStruct((_NC * _N, 16), jnp.float32),
        ],
        mesh=plsc.VectorSubcoreMesh(core_axis_name="c", subcore_axis_name="s"),
        scratch_types=[
            pltpu.VMEM((_NCH, _C), jnp.int32),
            pltpu.VMEM((2, _C), jnp.int32),
            pltpu.VMEM((2, _C), jnp.int32),
            pltpu.VMEM((_C, _D), jnp.float32),
            pltpu.VMEM((_C, _D), jnp.float32),
            pltpu.VMEM((_C, 16), jnp.float32),
            pltpu.VMEM((1, _TPT), jnp.int32),
            pltpu.VMEM((_TPT, 16), jnp.float32),
            pltpu.VMEM_SHARED((_N, _D), jnp.float32),
            pltpu.VMEM_SHARED((_N, 16), jnp.float32),
            pltpu.SemaphoreType.DMA,
            pltpu.SemaphoreType.DMA,
        ],
        compiler_params=pltpu.CompilerParams(use_tc_tiling_on_sc=False, skip_device_barrier=True),
    )
    feat_o, cnt_o = sc_call(pki, features, tid_p, twg, ones_c, zf, zc)

    out = pl.pallas_call(
        _tc_body,
        out_shape=jax.ShapeDtypeStruct((_N, _OUT), jnp.float32),
    )(feat_o, cnt_o, W_proj, expert_protos, W_expert, W_reg)
    return out


# speedup vs baseline: 1.3534x; 1.0235x over previous
import jax
import jax.numpy as jnp
from jax import lax
from jax.experimental import pallas as pl
from jax.experimental.pallas import tpu as pltpu
from jax.experimental.pallas import tpu_sc as plsc

_N = 10000
_D = 128
_E = 320000
_OUT = 64
_NC = 2
_NS = 16
_NW = _NC * _NS
_C = 80
_EPT = _E // _NW
_NCH = _EPT // _C
_G = _C // 16
_RPT = _N // _NS
_TPAD = 1024
_TPT = _TPAD // _NS


def _sc_body(pki_h, feat_h, tid_h, twg_h, ones_h, zc_h,
             feat_o, cnt_o,
             pki_v, sidxr, didxr, rows_a, rows_b, ones_v, tid_v, twg_v,
             accf, accc, sem_a, sem_b):
    cid = lax.axis_index("c")
    sid = lax.axis_index("s")
    wid = cid * _NS + sid
    r0 = sid * _RPT
    zv = jnp.zeros((16,), jnp.float32)

    def zrow(i, carry):
        for g in range(_D // 16):
            rows_a[i, pl.ds(16 * g, 16)] = zv
        return carry

    lax.fori_loop(0, _C, zrow, 0)
    for k in range(_RPT // _C):
        pltpu.sync_copy(rows_a, accf.at[pl.ds(r0 + _C * k, _C)])
    pltpu.sync_copy(rows_a.at[pl.ds(0, _RPT % _C)],
                    accf.at[pl.ds(r0 + (_RPT // _C) * _C, _RPT % _C)])
    pltpu.sync_copy(zc_h.at[pl.ds(r0, _RPT)], accc.at[pl.ds(r0, _RPT)])
    base = wid * _NCH
    pltpu.sync_copy(pki_h.at[pl.ds(base, _NCH)], pki_v)
    pltpu.sync_copy(ones_h, ones_v)
    pltpu.sync_copy(tid_h.at[pl.ds(sid * _TPT, _TPT)], tid_v.at[0])
    pltpu.sync_copy(twg_h.at[pl.ds(sid * _TPT, _TPT)], twg_v)
    plsc.subcore_barrier()

    def unpack(j, row):
        for g in range(_G):
            pk = pki_v[j, pl.ds(16 * g, 16)]
            sidxr[row, pl.ds(16 * g, 16)] = pk & 0xFFFF
            didxr[row, pl.ds(16 * g, 16)] = lax.shift_right_logical(pk, 16)

    unpack(0, 0)
    unpack(1, 1)
    pltpu.async_copy(feat_h.at[sidxr.at[0]], rows_a, sem_a)
    pltpu.async_copy(feat_h.at[sidxr.at[1]], rows_b, sem_b)

    def step(jj, carry):
        j2 = 2 * jj + 2
        j3 = 2 * jj + 3
        pltpu.make_async_copy(feat_h.at[sidxr.at[0]], rows_a, sem_a).wait()
        pltpu.sync_copy(rows_a, accf.at[didxr.at[0]], add=True)
        pltpu.sync_copy(ones_v, accc.at[didxr.at[0]], add=True)
        unpack(j2, 0)
        pltpu.async_copy(feat_h.at[sidxr.at[0]], rows_a, sem_a)
        pltpu.make_async_copy(feat_h.at[sidxr.at[1]], rows_b, sem_b).wait()
        pltpu.sync_copy(rows_b, accf.at[didxr.at[1]], add=True)
        pltpu.sync_copy(ones_v, accc.at[didxr.at[1]], add=True)

        @pl.when(j3 < _NCH)
        def _():
            unpack(j3, 1)
            pltpu.async_copy(feat_h.at[sidxr.at[1]], rows_b, sem_b)

        return carry

    lax.fori_loop(0, _NCH // 2, step, 0)
    pltpu.make_async_copy(feat_h.at[sidxr.at[0]], rows_a, sem_a).wait()
    pltpu.sync_copy(rows_a, accf.at[didxr.at[0]], add=True)
    pltpu.sync_copy(ones_v, accc.at[didxr.at[0]], add=True)
    pltpu.sync_copy(twg_v, accc.at[tid_v.at[0]], add=True)
    plsc.subcore_barrier()
    o0 = cid * _N + r0
    pltpu.sync_copy(accf.at[pl.ds(r0, _RPT)], feat_o.at[pl.ds(o0, _RPT)])
    pltpu.sync_copy(accc.at[pl.ds(r0, _RPT)], cnt_o.at[pl.ds(o0, _RPT)])


def _tc_body(fp, cp, wp, pr, we, wr, out):
    f = fp[...]
    c = cp[...]
    agg = f[:_N] + f[_N:]
    deg = jnp.maximum(c[:_N, 0:1] + c[_N:, 0:1], 1.0)
    nrm = agg / deg
    h = jnp.maximum(jnp.dot(nrm, wp[...], preferred_element_type=jnp.float32), 0.0)
    wv = c[:_N, 1:2] + c[_N:, 1:2]
    tpv = jnp.sum(h * wv, axis=0, keepdims=True) / jnp.sum(wv)
    diff = pr[...] - tpv
    d2 = jnp.sum(diff * diff, axis=1, keepdims=True)
    oh = (d2 == jnp.min(d2)).astype(jnp.float32)
    wsel = jnp.sum(we[...] * oh[:, :, None], axis=0)
    x = jnp.maximum(jnp.dot(nrm, wsel, preferred_element_type=jnp.float32), 0.0)
    out[...] = jnp.dot(x, wr[...], preferred_element_type=jnp.float32)


def kernel(features, edge_index, test_ids, W_proj, expert_protos, W_expert, W_reg):
    pki = (edge_index[0] | (edge_index[1] << 16)).reshape(_NW * _NCH, _C)
    ntest = test_ids.shape[0]
    tid_p = jnp.concatenate(
        [test_ids.astype(jnp.int32), jnp.zeros((_TPAD - ntest,), jnp.int32)])
    twg = jnp.zeros((_TPAD, 16), jnp.float32).at[:ntest, 1].set(1.0)
    ones_c = jnp.zeros((_C, 16), jnp.float32).at[:, 0].set(1.0)
    zc = jnp.zeros((_N, 16), jnp.float32)

    sc_call = pl.kernel(
        _sc_body,
        out_type=[
            jax.ShapeDtypeStruct((_NC * _N, _D), jnp.float32),
            jax.ShapeDtypeStruct((_NC * _N, 16), jnp.float32),
        ],
        mesh=plsc.VectorSubcoreMesh(core_axis_name="c", subcore_axis_name="s"),
        scratch_types=[
            pltpu.VMEM((_NCH, _C), jnp.int32),
            pltpu.VMEM((2, _C), jnp.int32),
            pltpu.VMEM((2, _C), jnp.int32),
            pltpu.VMEM((_C, _D), jnp.float32),
            pltpu.VMEM((_C, _D), jnp.float32),
            pltpu.VMEM((_C, 16), jnp.float32),
            pltpu.VMEM((1, _TPT), jnp.int32),
            pltpu.VMEM((_TPT, 16), jnp.float32),
            pltpu.VMEM_SHARED((_N, _D), jnp.float32),
            pltpu.VMEM_SHARED((_N, 16), jnp.float32),
            pltpu.SemaphoreType.DMA,
            pltpu.SemaphoreType.DMA,
        ],
        compiler_params=pltpu.CompilerParams(use_tc_tiling_on_sc=False),
    )
    feat_o, cnt_o = sc_call(pki, features, tid_p, twg, ones_c, zc)

    out = pl.pallas_call(
        _tc_body,
        out_shape=jax.ShapeDtypeStruct((_N, _OUT), jnp.float32),
    )(feat_o, cnt_o, W_proj, expert_protos, W_expert, W_reg)
    return out
